# baseline (speedup 1.0000x reference)
import jax
import jax.numpy as jnp
from jax.experimental import pallas as pl
from jax.experimental.pallas import tpu as pltpu

_BM = 2048

_NT = (((1,), (1,)), ((), ()))
_NN = (((1,), (0,)), ((), ()))


def _mlp_knots_kernel(x_ref, w1b_ref, w2b_ref, w3b_ref, out_ref):
    f32 = jnp.float32
    x = x_ref[0]
    bm = x.shape[0]
    xb = jnp.concatenate([x, jnp.ones((bm, 1), f32)], axis=1)

    h1 = jax.lax.dot_general(w1b_ref[...], xb, _NT,
                             preferred_element_type=f32)
    h1 = jnp.maximum(h1, 0.0)
    h1b = jnp.concatenate([h1, jnp.ones((1, bm), f32)], axis=0)

    h2 = jax.lax.dot_general(w2b_ref[...], h1b, _NN,
                             preferred_element_type=f32)
    h2 = jnp.maximum(h2, 0.0)
    h2b = jnp.concatenate([h2, jnp.ones((1, bm), f32)], axis=0)

    lg = jax.lax.dot_general(w3b_ref[...], h2b, _NN,
                             preferred_element_type=f32)
    m = jnp.max(lg, axis=0, keepdims=True)
    e = jnp.exp(lg - m)

    r16 = jax.lax.broadcasted_iota(jnp.int32, (16, 5), 0)
    k5 = jax.lax.broadcasted_iota(jnp.int32, (16, 5), 1)
    ct = (((r16 >= 4) & (r16 < 8) & (k5 <= (r16 - 4)))
          | ((r16 >= 8) & (r16 < 12))).astype(f32)

    u = jax.lax.dot_general(ct, e, _NN,
                            preferred_element_type=f32)
    u = u * (1.0 / u[8:9, :])
    ut = jnp.transpose(u, (1, 0))
    out_ref[...] = ut[:, :12]


@jax.jit
def kernel(input, W1, b1, W2, b2, W3, b3):
    B = input.shape[1]
    f32 = jnp.float32
    w1b = jnp.concatenate([W1, b1.reshape(-1, 1)], axis=1)
    w2b = jnp.concatenate([W2, b2.reshape(-1, 1)], axis=1)
    w3b = jnp.concatenate([W3, b3.reshape(-1, 1)], axis=1)

    out = pl.pallas_call(
        _mlp_knots_kernel,
        grid=(B // _BM,),
        in_specs=[
            pl.BlockSpec((1, _BM, 8), lambda i: (0, i, 0)),
            pl.BlockSpec((128, 9), lambda i: (0, 0)),
            pl.BlockSpec((128, 129), lambda i: (0, 0)),
            pl.BlockSpec((5, 129), lambda i: (0, 0)),
        ],
        out_specs=pl.BlockSpec((_BM, 12), lambda i: (i, 0)),
        out_shape=jax.ShapeDtypeStruct((B, 12), f32),
        compiler_params=pltpu.CompilerParams(
            dimension_semantics=("parallel",),
        ),
    )(input, w1b, w2b, w3b)
    return out


# speedup vs baseline: 1.0619x; 1.0619x over previous
import jax
import jax.numpy as jnp
from jax.experimental import pallas as pl
from jax.experimental.pallas import tpu as pltpu

_BM = 2048

_NT = (((1,), (1,)), ((), ()))
_NN = (((1,), (0,)), ((), ()))


def _mlp_knots_kernel(x_ref, w1b_ref, w2b_ref, w3b_ref, out_ref):
    f32 = jnp.float32
    x = x_ref[0]
    bm = x.shape[0]
    xb = jnp.concatenate([x, jnp.ones((bm, 1), f32)], axis=1)

    h1 = jax.lax.dot_general(w1b_ref[...], xb, _NT,
                             preferred_element_type=f32)
    h1 = jnp.maximum(h1, 0.0)
    h1b = jnp.concatenate([h1, jnp.ones((1, bm), f32)], axis=0)

    h2 = jax.lax.dot_general(w2b_ref[...], h1b, _NN,
                             preferred_element_type=f32)
    h2 = jnp.maximum(h2, 0.0)
    h2b = jnp.concatenate([h2, jnp.ones((1, bm), f32)], axis=0)

    lg = jax.lax.dot_general(w3b_ref[...], h2b, _NN,
                             preferred_element_type=f32)
    m = jnp.max(lg, axis=0, keepdims=True)
    e = jnp.exp(lg - m)

    r12 = jax.lax.broadcasted_iota(jnp.int32, (12, 5), 0)
    k5 = jax.lax.broadcasted_iota(jnp.int32, (12, 5), 1)
    ct = (((r12 >= 4) & (r12 < 8) & (k5 <= (r12 - 4)))
          | (r12 >= 8)).astype(f32)

    u = jax.lax.dot_general(ct, e, _NN,
                            preferred_element_type=f32)
    out_ref[...] = u * (1.0 / u[8:9, :])


@jax.jit
def kernel(input, W1, b1, W2, b2, W3, b3):
    B = input.shape[1]
    f32 = jnp.float32
    w1b = jnp.concatenate([W1, b1.reshape(-1, 1)], axis=1)
    w2b = jnp.concatenate([W2, b2.reshape(-1, 1)], axis=1)
    w3b = jnp.concatenate([W3, b3.reshape(-1, 1)], axis=1)

    out = pl.pallas_call(
        _mlp_knots_kernel,
        grid=(B // _BM,),
        in_specs=[
            pl.BlockSpec((1, _BM, 8), lambda i: (0, i, 0)),
            pl.BlockSpec((128, 9), lambda i: (0, 0)),
            pl.BlockSpec((128, 129), lambda i: (0, 0)),
            pl.BlockSpec((5, 129), lambda i: (0, 0)),
        ],
        out_specs=pl.BlockSpec((12, _BM), lambda i: (0, i)),
        out_shape=jax.ShapeDtypeStruct((12, B), f32),
        compiler_params=pltpu.CompilerParams(
            dimension_semantics=("parallel",),
        ),
    )(input, w1b, w2b, w3b)
    return jnp.stack([out[c] for c in range(12)], axis=1)
